# Initial kernel scaffold; baseline (speedup 1.0000x reference)
import jax
import jax.numpy as jnp
from jax.experimental import pallas as pl


def kernel(input_ids, embed0, embed1, mask_token_embed):
    raise NotImplementedError("write your pallas kernel here")


# speedup vs baseline: 4.2508x; 4.2508x over previous
import functools

import jax
import jax.numpy as jnp
from jax import lax
from jax.experimental import pallas as pl
from jax.experimental.pallas import tpu as pltpu
from jax.experimental.pallas import tpu_sc as plsc

L = 16
NC = 2
NS = 16
NW = NC * NS
D = 256
V = 512
MASK_ID = V * V
N_TOK = 4 * 32 * 1024
TPW = N_TOK // NW
C = 128
NCHUNK = TPW // C


def _make_sc_embed():
    mesh = plsc.VectorSubcoreMesh(core_axis_name="c", subcore_axis_name="s")

    @functools.partial(
        pl.kernel,
        out_type=jax.ShapeDtypeStruct((N_TOK, D), jnp.float32),
        mesh=mesh,
        scratch_types=[
            pltpu.VMEM((TPW,), jnp.int32),
            pltpu.VMEM((TPW,), jnp.int32),
            pltpu.VMEM((TPW,), jnp.int32),
            pltpu.VMEM((C, D), jnp.float32),
            pltpu.VMEM((C, D), jnp.float32),
            pltpu.SemaphoreType.DMA,
            pltpu.SemaphoreType.DMA,
        ],
    )
    def sc_embed(ids_hbm, t0_hbm, t1_hbm, out_hbm,
                 ids_v, idx0_v, idx1_v, buf0, buf1, sem0, sem1):
        wid = lax.axis_index("s") * NC + lax.axis_index("c")
        base = wid * TPW
        pltpu.sync_copy(ids_hbm.at[pl.ds(base, TPW)], ids_v)

        @pl.loop(0, TPW // L)
        def _(g):
            sl = pl.ds(g * L, L)
            ids16 = ids_v[sl]
            is_mask = ids16 == MASK_ID
            idx0_v[sl] = jnp.where(is_mask, V, lax.bitwise_and(ids16, V - 1))
            idx1_v[sl] = jnp.where(
                is_mask, V,
                lax.bitwise_and(lax.shift_right_logical(ids16, 9), V - 1))

        @pl.loop(0, NCHUNK)
        def _(j):
            tok = j * C
            cp0 = pltpu.async_copy(t0_hbm.at[idx0_v.at[pl.ds(tok, C)]], buf0, sem0)
            cp1 = pltpu.async_copy(t1_hbm.at[idx1_v.at[pl.ds(tok, C)]], buf1, sem1)
            cp0.wait()
            cp1.wait()

            @pl.loop(0, C)
            def _(r):
                for k in range(D // L):
                    sl = pl.ds(k * L, L)
                    buf0[r, sl] = buf0[r, sl] + buf1[r, sl]

            pltpu.sync_copy(buf0, out_hbm.at[pl.ds(base + tok, C)])

    return sc_embed


_SC_EMBED = _make_sc_embed()


def kernel(input_ids, embed0, embed1, mask_token_embed):
    ids = input_ids.reshape(N_TOK)
    t0 = jnp.concatenate([embed0, mask_token_embed], axis=0)
    t1 = jnp.concatenate([embed1, jnp.zeros((1, D), jnp.float32)], axis=0)
    out = _SC_EMBED(ids, t0, t1)
    return out.reshape(*input_ids.shape, D)


# speedup vs baseline: 5.1840x; 1.2196x over previous
import functools

import jax
import jax.numpy as jnp
from jax import lax
from jax.experimental import pallas as pl
from jax.experimental.pallas import tpu as pltpu
from jax.experimental.pallas import tpu_sc as plsc

L = 16
NC = 2
NS = 16
NW = NC * NS
D = 256
V = 512
MASK_ID = V * V
N_TOK = 4 * 32 * 1024
TPW = N_TOK // NW
C = 64
NCHUNK = TPW // C


def _make_sc_embed():
    mesh = plsc.VectorSubcoreMesh(core_axis_name="c", subcore_axis_name="s")

    @functools.partial(
        pl.kernel,
        out_type=jax.ShapeDtypeStruct((N_TOK, D), jnp.float32),
        mesh=mesh,
        scratch_types=[
            pltpu.VMEM((TPW,), jnp.int32),
            pltpu.VMEM((TPW,), jnp.int32),
            pltpu.VMEM((TPW,), jnp.int32),
            pltpu.VMEM((C, D), jnp.float32),
            pltpu.VMEM((C, D), jnp.float32),
            pltpu.VMEM((C, D), jnp.float32),
            pltpu.VMEM((C, D), jnp.float32),
            pltpu.VMEM((C, D), jnp.float32),
            pltpu.VMEM((C, D), jnp.float32),
            pltpu.SemaphoreType.DMA,
            pltpu.SemaphoreType.DMA,
            pltpu.SemaphoreType.DMA,
            pltpu.SemaphoreType.DMA,
            pltpu.SemaphoreType.DMA,
            pltpu.SemaphoreType.DMA,
        ],
    )
    def sc_embed(ids_hbm, t0_hbm, t1_hbm, out_hbm,
                 ids_v, idx0_v, idx1_v,
                 g0a, g1a, oa, g0b, g1b, ob,
                 sg0a, sg1a, swa, sg0b, sg1b, swb):
        wid = lax.axis_index("s") * NC + lax.axis_index("c")
        base = wid * TPW
        pltpu.sync_copy(ids_hbm.at[pl.ds(base, TPW)], ids_v)

        @pl.loop(0, TPW // L)
        def _(g):
            sl = pl.ds(g * L, L)
            ids16 = ids_v[sl]
            is_mask = ids16 == MASK_ID
            idx0_v[sl] = jnp.where(is_mask, V, lax.bitwise_and(ids16, V - 1))
            idx1_v[sl] = jnp.where(
                is_mask, V,
                lax.bitwise_and(lax.shift_right_logical(ids16, 9), V - 1))

        sets = ((g0a, g1a, oa, sg0a, sg1a, swa),
                (g0b, g1b, ob, sg0b, sg1b, swb))

        def gather_descs(i, st):
            g0, g1, _, s0, s1, _ = st
            c0 = pltpu.make_async_copy(
                t0_hbm.at[idx0_v.at[pl.ds(i * C, C)]], g0, s0)
            c1 = pltpu.make_async_copy(
                t1_hbm.at[idx1_v.at[pl.ds(i * C, C)]], g1, s1)
            return c0, c1

        for b in range(2):
            for c in gather_descs(b, sets[b]):
                c.start()

        @pl.loop(0, NCHUNK, step=2)
        def _(j):
            for b in range(2):
                i = j + b
                g0, g1, o, s0, s1, sw = sets[b]
                for c in gather_descs(i, sets[b]):
                    c.wait()

                @pl.when(j > 0)
                def _():
                    pltpu.make_async_copy(
                        o, out_hbm.at[pl.ds(base + (i - 2) * C, C)], sw).wait()

                @pl.loop(0, C)
                def _(r):
                    for k in range(D // L):
                        sl = pl.ds(k * L, L)
                        o[r, sl] = g0[r, sl] + g1[r, sl]

                pltpu.async_copy(o, out_hbm.at[pl.ds(base + i * C, C)], sw)

                @pl.when(j < NCHUNK - 2)
                def _():
                    for c in gather_descs(i + 2, sets[b]):
                        c.start()

        for b in range(2):
            i = NCHUNK - 2 + b
            _, _, o, _, _, sw = sets[b]
            pltpu.make_async_copy(
                o, out_hbm.at[pl.ds(base + i * C, C)], sw).wait()

    return sc_embed


_SC_EMBED = _make_sc_embed()


def kernel(input_ids, embed0, embed1, mask_token_embed):
    ids = input_ids.reshape(N_TOK)
    t0 = jnp.concatenate([embed0, mask_token_embed], axis=0)
    t1 = jnp.concatenate([embed1, jnp.zeros((1, D), jnp.float32)], axis=0)
    out = _SC_EMBED(ids, t0, t1)
    return out.reshape(*input_ids.shape, D)
